# baseline (speedup 1.0000x reference)
import functools

import jax
import jax.numpy as jnp
from jax import lax
from jax.experimental import pallas as pl
from jax.experimental.pallas import tpu as pltpu
from jax.experimental.pallas import tpu_sc as plsc

N_NODES = 10000
N_HEDGES = 5000
NNZ = 320000
D = 128
W = 64

NC = 2
NS = 16
NW = NC * NS
L = 16

CH = 128
EPT = NNZ // NS
NCHUNK = 159
TOTCH = 160
NCH_H = 80
N_PAD = 10240
E_PAD = 5120
PAD_NODE = N_PAD - 1
PAD_HEDGE = E_PAD - 1
RPE = E_PAD // NS
RPN = N_PAD // NS

_mesh = functools.partial(
    plsc.VectorSubcoreMesh, core_axis_name="c", subcore_axis_name="s",
    num_cores=NC, num_subcores=NS)
_sc_params = pltpu.CompilerParams(needs_layout_passes=False,
                                  use_tc_tiling_on_sc=False)


def _hist_body(node_hbm, hedge_hbm, zeros_hbm, dvp_hbm, dep_hbm,
               nidx_v, hidx_v, histn_v, histe_v):
    c = lax.axis_index("c")
    s = lax.axis_index("s")
    w = s * NC + c
    pltpu.sync_copy(zeros_hbm, histn_v)
    pltpu.sync_copy(zeros_hbm.at[pl.ds(0, E_PAD // L)], histe_v)
    pltpu.sync_copy(node_hbm.at[s, pl.ds(c * NCH_H, NCH_H)], nidx_v)
    pltpu.sync_copy(hedge_hbm.at[s, pl.ds(c * NCH_H, NCH_H)], hidx_v)
    ones = jnp.full((L,), 1.0, jnp.float32)

    def body(j, carry):
        for q in range(CH // L):
            col = q * L
            nv = nidx_v[j, pl.ds(col, L)]
            plsc.addupdate_scatter(histn_v, [nv >> 4, nv & 15], ones)
            hv = hidx_v[j, pl.ds(col, L)]
            plsc.addupdate_scatter(histe_v, [hv >> 4, hv & 15], ones)
        return carry

    lax.fori_loop(0, NCH_H, body, 0)
    pltpu.sync_copy(histn_v, dvp_hbm.at[w])
    pltpu.sync_copy(histe_v, dep_hbm.at[w])


def _hist_call(node_p, hedge_p, zeros1):
    return pl.kernel(
        _hist_body,
        out_type=(jax.ShapeDtypeStruct((NW, N_PAD // L, L), jnp.float32),
                  jax.ShapeDtypeStruct((NW, E_PAD // L, L), jnp.float32)),
        mesh=_mesh(),
        compiler_params=_sc_params,
        scratch_types=[
            pltpu.VMEM((NCH_H, CH), jnp.int32),
            pltpu.VMEM((NCH_H, CH), jnp.int32),
            pltpu.VMEM((N_PAD // L, L), jnp.float32),
            pltpu.VMEM((E_PAD // L, L), jnp.float32),
        ],
    )(node_p, hedge_p, zeros1)


ZB = 64


def _main_body(y_hbm, dem_hbm, nidx_hbm, hidx_hbm, zeros_hbm, out_hbm,
               nidx_v, hidx_v, bufa, bufb, bufc, xe_sh, ya_sh,
               gsa, gsb, gsc, ssa, ssb, ssc):
    c = lax.axis_index("c")
    s = lax.axis_index("s")
    ba = bufa.at[pl.ds(0, ZB)]
    bb = bufb.at[pl.ds(0, ZB)]

    pltpu.async_copy(nidx_hbm.at[s], nidx_v, gsa)
    pltpu.async_copy(hidx_hbm.at[s], hidx_v, gsb)
    for k in range(RPN // ZB):
        r0 = s * RPN + k * ZB
        pltpu.async_copy(y_hbm.at[c, pl.ds(r0, ZB)], ya_sh.at[pl.ds(r0, ZB)],
                         gsc)
    pltpu.sync_copy(zeros_hbm, ba)
    for k in range(RPE // ZB):
        pltpu.async_copy(ba, xe_sh.at[pl.ds(s * RPE + k * ZB, ZB)], ssa)
    for k in range(RPN // ZB):
        r0 = s * RPN + k * ZB
        pltpu.make_async_copy(y_hbm.at[c, pl.ds(r0, ZB)],
                              ya_sh.at[pl.ds(r0, ZB)], gsc).wait()
    for k in range(RPE // ZB):
        pltpu.make_async_copy(ba, xe_sh.at[pl.ds(s * RPE + k * ZB, ZB)],
                              ssa).wait()
    pltpu.make_async_copy(nidx_hbm.at[s], nidx_v, gsa).wait()
    pltpu.make_async_copy(hidx_hbm.at[s], hidx_v, gsb).wait()
    plsc.subcore_barrier()

    def sweep(src_sh, dst_sh, gidx_v, sidx_v):
        bufs = (bufa, bufb, bufc)
        gs = (gsa, gsb, gsc)
        ss = (ssa, ssb, ssc)

        def g(j, k):
            pltpu.async_copy(src_sh.at[gidx_v.at[j]], bufs[k], gs[k])

        def gwait(j, k):
            pltpu.make_async_copy(src_sh.at[gidx_v.at[j]], bufs[k],
                                  gs[k]).wait()

        def sct(j, k):
            pltpu.async_copy(bufs[k], dst_sh.at[sidx_v.at[j]], ss[k],
                             add=True)

        def swait(j, k):
            pltpu.make_async_copy(bufs[k], dst_sh.at[sidx_v.at[j]],
                                  ss[k]).wait()

        g(0, 0)
        g(1, 1)
        gwait(0, 0)
        sct(0, 0)
        g(2, 2)

        def step(j, carry):
            for k in range(3):
                @pl.when(j % 3 == k)
                def _():
                    gwait(j, k)
                    sct(j, k)
                    swait(j - 1, (k + 2) % 3)
                    g(j + 2, (k + 2) % 3)
            return carry

        lax.fori_loop(1, NCHUNK - 1, step, 0)
        j = NCHUNK - 1
        gwait(j, j % 3)
        sct(j, j % 3)
        swait(j - 1, (j - 1) % 3)
        gwait(NCHUNK, NCHUNK % 3)
        swait(j, j % 3)

    sweep(ya_sh, xe_sh, nidx_v, hidx_v)
    plsc.subcore_barrier()

    def scale(r, carry):
        for col in range(0, W, L):
            bufa[r, pl.ds(col, L)] = (bufa[r, pl.ds(col, L)]
                                      * bufb[r, pl.ds(col, L)])
        return carry

    for k in range(RPE // ZB):
        e0 = s * RPE + k * ZB
        pltpu.sync_copy(xe_sh.at[pl.ds(e0, ZB)], ba)
        pltpu.sync_copy(dem_hbm.at[pl.ds(e0, ZB)], bb)
        lax.fori_loop(0, ZB, scale, 0)
        pltpu.sync_copy(ba, xe_sh.at[pl.ds(e0, ZB)])

    pltpu.sync_copy(zeros_hbm, ba)
    for k in range(RPN // ZB):
        pltpu.async_copy(ba, ya_sh.at[pl.ds(s * RPN + k * ZB, ZB)], ssa)
    for k in range(RPN // ZB):
        pltpu.make_async_copy(ba, ya_sh.at[pl.ds(s * RPN + k * ZB, ZB)],
                              ssa).wait()
    plsc.subcore_barrier()

    sweep(xe_sh, ya_sh, hidx_v, nidx_v)
    plsc.subcore_barrier()

    for k in range(RPN // ZB):
        n0 = s * RPN + k * ZB
        stg = (bufa, bufb)[k % 2].at[pl.ds(0, ZB)]
        sem = (gsa, gsb)[k % 2]
        if k >= 2:
            p0 = s * RPN + (k - 2) * ZB
            pltpu.make_async_copy(stg, out_hbm.at[c, pl.ds(p0, ZB)],
                                  sem).wait()
        pltpu.sync_copy(ya_sh.at[pl.ds(n0, ZB)], stg)
        pltpu.async_copy(stg, out_hbm.at[c, pl.ds(n0, ZB)], sem)
    for k in range(RPN // ZB - 2, RPN // ZB):
        n0 = s * RPN + k * ZB
        stg = (bufa, bufb)[k % 2].at[pl.ds(0, ZB)]
        sem = (gsa, gsb)[k % 2]
        pltpu.make_async_copy(stg, out_hbm.at[c, pl.ds(n0, ZB)], sem).wait()


def _main_call(ystack, demat, nidx, hidx):
    zeros2 = jnp.zeros((ZB, W), jnp.float32)
    return pl.kernel(
        _main_body,
        out_type=jax.ShapeDtypeStruct((NC, N_PAD, W), jnp.float32),
        mesh=_mesh(),
        compiler_params=_sc_params,
        scratch_types=[
            pltpu.VMEM((TOTCH, CH), jnp.int32),
            pltpu.VMEM((TOTCH, CH), jnp.int32),
            pltpu.VMEM((CH, W), jnp.float32),
            pltpu.VMEM((CH, W), jnp.float32),
            pltpu.VMEM((CH, W), jnp.float32),
            pltpu.VMEM_SHARED((E_PAD, W), jnp.float32),
            pltpu.VMEM_SHARED((N_PAD, W), jnp.float32),
            pltpu.SemaphoreType.DMA,
            pltpu.SemaphoreType.DMA,
            pltpu.SemaphoreType.DMA,
            pltpu.SemaphoreType.DMA,
            pltpu.SemaphoreType.DMA,
            pltpu.SemaphoreType.DMA,
        ],
    )(ystack, demat, nidx, hidx, zeros2)


def _tc1_body(feat_ref, win_ref, bin_ref, w2_ref, b2_ref, dvp_ref, dep_ref,
              xt_ref, y_ref, dem_ref):
    x = feat_ref[...]
    h = lax.dot_general(x, win_ref[...], (((1,), (1,)), ((), ())),
                        preferred_element_type=jnp.float32) + bin_ref[...]
    h = jnp.where(h >= 0, h, 0.2 * h)
    xt = lax.dot_general(h, w2_ref[...], (((1,), (1,)), ((), ())),
                         preferred_element_type=jnp.float32) + b2_ref[...]
    dv = jnp.sum(dvp_ref[...], axis=0)
    dvis = jnp.where(dv > 0, lax.rsqrt(dv), 0.0)
    xt_ref[...] = xt
    y = xt * dvis[:, None]
    y_ref[0] = y[:, :W]
    y_ref[1] = y[:, W:]
    de = jnp.sum(dep_ref[...], axis=0)
    deinv = jnp.where(de > 0, 1.0 / de, 0.0)
    dem_ref[...] = jnp.broadcast_to(deinv[:, None], (E_PAD // 10, W))


def _tc1_call(feat_p, w_in, b_in, w2, b2, dvp, dep):
    blk = 1024
    eblk = E_PAD // 10
    grid = N_PAD // blk
    return pl.pallas_call(
        _tc1_body,
        grid=(grid,),
        in_specs=[
            pl.BlockSpec((blk, D), lambda i: (i, 0)),
            pl.BlockSpec((D, D), lambda i: (0, 0)),
            pl.BlockSpec((1, D), lambda i: (0, 0)),
            pl.BlockSpec((D, D), lambda i: (0, 0)),
            pl.BlockSpec((1, D), lambda i: (0, 0)),
            pl.BlockSpec((NW, blk), lambda i: (0, i)),
            pl.BlockSpec((NW, eblk), lambda i: (0, i)),
        ],
        out_specs=[pl.BlockSpec((blk, D), lambda i: (i, 0)),
                   pl.BlockSpec((NC, blk, W), lambda i: (0, i, 0)),
                   pl.BlockSpec((eblk, W), lambda i: (i, 0))],
        out_shape=[jax.ShapeDtypeStruct((N_PAD, D), jnp.float32),
                   jax.ShapeDtypeStruct((NC, N_PAD, W), jnp.float32),
                   jax.ShapeDtypeStruct((E_PAD, W), jnp.float32)],
    )(feat_p, w_in, b_in, w2, b2, dvp, dep)


def _tc3_body(pn_ref, dvp_ref, xt_ref, o_ref):
    p = jnp.concatenate([pn_ref[0], pn_ref[1]], axis=1)
    dv = jnp.sum(dvp_ref[...], axis=0)
    dvis = jnp.where(dv > 0, lax.rsqrt(dv), 0.0)
    o_ref[...] = jnp.maximum(p * dvis[:, None] + xt_ref[...], 0.0)


def _tc3_call(pn, dvp, xt):
    blk = 1024
    grid = N_PAD // blk
    return pl.pallas_call(
        _tc3_body,
        grid=(grid,),
        in_specs=[
            pl.BlockSpec((NC, blk, W), lambda i: (0, i, 0)),
            pl.BlockSpec((NW, blk), lambda i: (0, i)),
            pl.BlockSpec((blk, D), lambda i: (i, 0)),
        ],
        out_specs=pl.BlockSpec((blk, D), lambda i: (i, 0)),
        out_shape=jax.ShapeDtypeStruct((N_PAD, D), jnp.float32),
    )(pn, dvp, xt)[:N_NODES]


def _pad_idx(idx, pad_val):
    cols = TOTCH * CH - EPT
    return jnp.concatenate(
        [idx.reshape(NS, EPT),
         jnp.full((NS, cols), pad_val, jnp.int32)],
        axis=1).reshape(NS, TOTCH, CH)


def kernel(feat, node_idx, hedge_idx, W_in, b_in, W1, b1, W2, b2):
    f32 = jnp.float32
    feat_p = jnp.zeros((N_PAD, D), f32).at[:N_NODES, :].set(feat)
    node_p = _pad_idx(node_idx, PAD_NODE)
    hedge_p = _pad_idx(hedge_idx, PAD_HEDGE)
    zeros1 = jnp.zeros((N_PAD // L, L), f32)

    dvp, dep = _hist_call(node_p, hedge_p, zeros1)
    dvp = dvp.reshape(NW, N_PAD)
    dep = dep.reshape(NW, E_PAD)
    xt, ystack, demat = _tc1_call(feat_p, W_in, b_in.reshape(1, D), W2,
                                  b2.reshape(1, D), dvp, dep)
    pn = _main_call(ystack, demat, node_p, hedge_p)
    return _tc3_call(pn, dvp, xt)


# speedup vs baseline: 1.0253x; 1.0253x over previous
import functools

import jax
import jax.numpy as jnp
from jax import lax
from jax.experimental import pallas as pl
from jax.experimental.pallas import tpu as pltpu
from jax.experimental.pallas import tpu_sc as plsc

N_NODES = 10000
N_HEDGES = 5000
NNZ = 320000
D = 128
W = 64

NC = 2
NS = 16
NW = NC * NS
L = 16

CH = 96
EPT = NNZ // NS
NCHUNK = 209
TOTCH = 224
NCH_H = 112
N_PAD = 10240
E_PAD = 5120
PAD_NODE = N_PAD - 1
PAD_HEDGE = E_PAD - 1
RPE = E_PAD // NS
RPN = N_PAD // NS

_mesh = functools.partial(
    plsc.VectorSubcoreMesh, core_axis_name="c", subcore_axis_name="s",
    num_cores=NC, num_subcores=NS)
_sc_params = pltpu.CompilerParams(needs_layout_passes=False,
                                  use_tc_tiling_on_sc=False)


def _hist_body(node_hbm, hedge_hbm, zeros_hbm, dvp_hbm, dep_hbm,
               nidx_v, hidx_v, histn_v, histe_v):
    c = lax.axis_index("c")
    s = lax.axis_index("s")
    w = s * NC + c
    pltpu.sync_copy(zeros_hbm, histn_v)
    pltpu.sync_copy(zeros_hbm.at[pl.ds(0, E_PAD // L)], histe_v)
    pltpu.sync_copy(node_hbm.at[s, pl.ds(c * NCH_H, NCH_H)], nidx_v)
    pltpu.sync_copy(hedge_hbm.at[s, pl.ds(c * NCH_H, NCH_H)], hidx_v)
    ones = jnp.full((L,), 1.0, jnp.float32)

    def body(j, carry):
        for q in range(CH // L):
            col = q * L
            nv = nidx_v[j, pl.ds(col, L)]
            plsc.addupdate_scatter(histn_v, [nv >> 4, nv & 15], ones)
            hv = hidx_v[j, pl.ds(col, L)]
            plsc.addupdate_scatter(histe_v, [hv >> 4, hv & 15], ones)
        return carry

    lax.fori_loop(0, NCH_H, body, 0)
    pltpu.sync_copy(histn_v, dvp_hbm.at[w])
    pltpu.sync_copy(histe_v, dep_hbm.at[w])


def _hist_call(node_p, hedge_p, zeros1):
    return pl.kernel(
        _hist_body,
        out_type=(jax.ShapeDtypeStruct((NW, N_PAD // L, L), jnp.float32),
                  jax.ShapeDtypeStruct((NW, E_PAD // L, L), jnp.float32)),
        mesh=_mesh(),
        compiler_params=_sc_params,
        scratch_types=[
            pltpu.VMEM((NCH_H, CH), jnp.int32),
            pltpu.VMEM((NCH_H, CH), jnp.int32),
            pltpu.VMEM((N_PAD // L, L), jnp.float32),
            pltpu.VMEM((E_PAD // L, L), jnp.float32),
        ],
    )(node_p, hedge_p, zeros1)


ZB = 64


def _main_body(y_hbm, dem_hbm, nidx_hbm, hidx_hbm, zeros_hbm, out_hbm,
               nidx_v, hidx_v, bufa, bufb, bufc, bufd, xe_sh, ya_sh,
               gsa, gsb, gsc, gsd, ssa, ssb, ssc, ssd):
    c = lax.axis_index("c")
    s = lax.axis_index("s")
    ba = bufa.at[pl.ds(0, ZB)]
    bb = bufb.at[pl.ds(0, ZB)]

    pltpu.async_copy(nidx_hbm.at[s], nidx_v, gsa)
    pltpu.async_copy(hidx_hbm.at[s], hidx_v, gsb)
    for k in range(RPN // ZB):
        r0 = s * RPN + k * ZB
        pltpu.async_copy(y_hbm.at[c, pl.ds(r0, ZB)], ya_sh.at[pl.ds(r0, ZB)],
                         gsc)
    pltpu.sync_copy(zeros_hbm, ba)
    for k in range(RPE // ZB):
        pltpu.async_copy(ba, xe_sh.at[pl.ds(s * RPE + k * ZB, ZB)], ssa)
    for k in range(RPN // ZB):
        r0 = s * RPN + k * ZB
        pltpu.make_async_copy(y_hbm.at[c, pl.ds(r0, ZB)],
                              ya_sh.at[pl.ds(r0, ZB)], gsc).wait()
    for k in range(RPE // ZB):
        pltpu.make_async_copy(ba, xe_sh.at[pl.ds(s * RPE + k * ZB, ZB)],
                              ssa).wait()
    pltpu.make_async_copy(nidx_hbm.at[s], nidx_v, gsa).wait()
    pltpu.make_async_copy(hidx_hbm.at[s], hidx_v, gsb).wait()
    plsc.subcore_barrier()

    def sweep(src_sh, dst_sh, gidx_v, sidx_v):
        bufs = (bufa, bufb, bufc, bufd)
        gs = (gsa, gsb, gsc, gsd)
        ss = (ssa, ssb, ssc, ssd)

        def g(j, k):
            pltpu.async_copy(src_sh.at[gidx_v.at[j]], bufs[k], gs[k])

        def gwait(j, k):
            pltpu.make_async_copy(src_sh.at[gidx_v.at[j]], bufs[k],
                                  gs[k]).wait()

        def sct(j, k):
            pltpu.async_copy(bufs[k], dst_sh.at[sidx_v.at[j]], ss[k],
                             add=True)

        def swait(j, k):
            pltpu.make_async_copy(bufs[k], dst_sh.at[sidx_v.at[j]],
                                  ss[k]).wait()

        g(0, 0)
        g(1, 1)
        gwait(0, 0)
        sct(0, 0)
        g(2, 2)
        gwait(1, 1)
        sct(1, 1)
        g(3, 3)

        def step(j, carry):
            for k in range(4):
                @pl.when(j % 4 == k)
                def _():
                    gwait(j, k)
                    sct(j, k)
                    swait(j - 2, (k + 2) % 4)
                    g(j + 2, (k + 2) % 4)
            return carry

        lax.fori_loop(2, NCHUNK - 1, step, 0)
        j = NCHUNK - 1
        gwait(j, j % 4)
        sct(j, j % 4)
        swait(j - 2, (j - 2) % 4)
        swait(j - 1, (j - 1) % 4)
        gwait(NCHUNK, NCHUNK % 4)
        swait(j, j % 4)

    sweep(ya_sh, xe_sh, nidx_v, hidx_v)
    plsc.subcore_barrier()

    def scale(r, carry):
        for col in range(0, W, L):
            bufa[r, pl.ds(col, L)] = (bufa[r, pl.ds(col, L)]
                                      * bufb[r, pl.ds(col, L)])
        return carry

    for k in range(RPE // ZB):
        e0 = s * RPE + k * ZB
        pltpu.sync_copy(xe_sh.at[pl.ds(e0, ZB)], ba)
        pltpu.sync_copy(dem_hbm.at[pl.ds(e0, ZB)], bb)
        lax.fori_loop(0, ZB, scale, 0)
        pltpu.sync_copy(ba, xe_sh.at[pl.ds(e0, ZB)])

    pltpu.sync_copy(zeros_hbm, ba)
    for k in range(RPN // ZB):
        pltpu.async_copy(ba, ya_sh.at[pl.ds(s * RPN + k * ZB, ZB)], ssa)
    for k in range(RPN // ZB):
        pltpu.make_async_copy(ba, ya_sh.at[pl.ds(s * RPN + k * ZB, ZB)],
                              ssa).wait()
    plsc.subcore_barrier()

    sweep(xe_sh, ya_sh, hidx_v, nidx_v)
    plsc.subcore_barrier()

    for k in range(RPN // ZB):
        n0 = s * RPN + k * ZB
        stg = (bufa, bufb)[k % 2].at[pl.ds(0, ZB)]
        sem = (gsa, gsb)[k % 2]
        if k >= 2:
            p0 = s * RPN + (k - 2) * ZB
            pltpu.make_async_copy(stg, out_hbm.at[c, pl.ds(p0, ZB)],
                                  sem).wait()
        pltpu.sync_copy(ya_sh.at[pl.ds(n0, ZB)], stg)
        pltpu.async_copy(stg, out_hbm.at[c, pl.ds(n0, ZB)], sem)
    for k in range(RPN // ZB - 2, RPN // ZB):
        n0 = s * RPN + k * ZB
        stg = (bufa, bufb)[k % 2].at[pl.ds(0, ZB)]
        sem = (gsa, gsb)[k % 2]
        pltpu.make_async_copy(stg, out_hbm.at[c, pl.ds(n0, ZB)], sem).wait()


def _main_call(ystack, demat, nidx, hidx):
    zeros2 = jnp.zeros((ZB, W), jnp.float32)
    return pl.kernel(
        _main_body,
        out_type=jax.ShapeDtypeStruct((NC, N_PAD, W), jnp.float32),
        mesh=_mesh(),
        compiler_params=_sc_params,
        scratch_types=[
            pltpu.VMEM((TOTCH, CH), jnp.int32),
            pltpu.VMEM((TOTCH, CH), jnp.int32),
            pltpu.VMEM((CH, W), jnp.float32),
            pltpu.VMEM((CH, W), jnp.float32),
            pltpu.VMEM((CH, W), jnp.float32),
            pltpu.VMEM((CH, W), jnp.float32),
            pltpu.VMEM_SHARED((E_PAD, W), jnp.float32),
            pltpu.VMEM_SHARED((N_PAD, W), jnp.float32),
            pltpu.SemaphoreType.DMA,
            pltpu.SemaphoreType.DMA,
            pltpu.SemaphoreType.DMA,
            pltpu.SemaphoreType.DMA,
            pltpu.SemaphoreType.DMA,
            pltpu.SemaphoreType.DMA,
            pltpu.SemaphoreType.DMA,
            pltpu.SemaphoreType.DMA,
        ],
    )(ystack, demat, nidx, hidx, zeros2)


def _tc1_body(feat_ref, win_ref, bin_ref, w2_ref, b2_ref, dvp_ref, dep_ref,
              xt_ref, y_ref, dem_ref):
    x = feat_ref[...]
    h = lax.dot_general(x, win_ref[...], (((1,), (1,)), ((), ())),
                        preferred_element_type=jnp.float32) + bin_ref[...]
    h = jnp.where(h >= 0, h, 0.2 * h)
    xt = lax.dot_general(h, w2_ref[...], (((1,), (1,)), ((), ())),
                         preferred_element_type=jnp.float32) + b2_ref[...]
    dv = jnp.sum(dvp_ref[...], axis=0)
    dvis = jnp.where(dv > 0, lax.rsqrt(dv), 0.0)
    xt_ref[...] = xt
    y = xt * dvis[:, None]
    y_ref[0] = y[:, :W]
    y_ref[1] = y[:, W:]
    de = jnp.sum(dep_ref[...], axis=0)
    deinv = jnp.where(de > 0, 1.0 / de, 0.0)
    dem_ref[...] = jnp.broadcast_to(deinv[:, None], (E_PAD // 10, W))


def _tc1_call(feat_p, w_in, b_in, w2, b2, dvp, dep):
    blk = 1024
    eblk = E_PAD // 10
    grid = N_PAD // blk
    return pl.pallas_call(
        _tc1_body,
        grid=(grid,),
        in_specs=[
            pl.BlockSpec((blk, D), lambda i: (i, 0)),
            pl.BlockSpec((D, D), lambda i: (0, 0)),
            pl.BlockSpec((1, D), lambda i: (0, 0)),
            pl.BlockSpec((D, D), lambda i: (0, 0)),
            pl.BlockSpec((1, D), lambda i: (0, 0)),
            pl.BlockSpec((NW, blk), lambda i: (0, i)),
            pl.BlockSpec((NW, eblk), lambda i: (0, i)),
        ],
        out_specs=[pl.BlockSpec((blk, D), lambda i: (i, 0)),
                   pl.BlockSpec((NC, blk, W), lambda i: (0, i, 0)),
                   pl.BlockSpec((eblk, W), lambda i: (i, 0))],
        out_shape=[jax.ShapeDtypeStruct((N_PAD, D), jnp.float32),
                   jax.ShapeDtypeStruct((NC, N_PAD, W), jnp.float32),
                   jax.ShapeDtypeStruct((E_PAD, W), jnp.float32)],
    )(feat_p, w_in, b_in, w2, b2, dvp, dep)


def _tc3_body(pn_ref, dvp_ref, xt_ref, o_ref):
    p = jnp.concatenate([pn_ref[0], pn_ref[1]], axis=1)
    dv = jnp.sum(dvp_ref[...], axis=0)
    dvis = jnp.where(dv > 0, lax.rsqrt(dv), 0.0)
    o_ref[...] = jnp.maximum(p * dvis[:, None] + xt_ref[...], 0.0)


def _tc3_call(pn, dvp, xt):
    blk = 1024
    grid = N_PAD // blk
    return pl.pallas_call(
        _tc3_body,
        grid=(grid,),
        in_specs=[
            pl.BlockSpec((NC, blk, W), lambda i: (0, i, 0)),
            pl.BlockSpec((NW, blk), lambda i: (0, i)),
            pl.BlockSpec((blk, D), lambda i: (i, 0)),
        ],
        out_specs=pl.BlockSpec((blk, D), lambda i: (i, 0)),
        out_shape=jax.ShapeDtypeStruct((N_PAD, D), jnp.float32),
    )(pn, dvp, xt)[:N_NODES]


def _pad_idx(idx, pad_val):
    cols = TOTCH * CH - EPT
    return jnp.concatenate(
        [idx.reshape(NS, EPT),
         jnp.full((NS, cols), pad_val, jnp.int32)],
        axis=1).reshape(NS, TOTCH, CH)


def kernel(feat, node_idx, hedge_idx, W_in, b_in, W1, b1, W2, b2):
    f32 = jnp.float32
    feat_p = jnp.zeros((N_PAD, D), f32).at[:N_NODES, :].set(feat)
    node_p = _pad_idx(node_idx, PAD_NODE)
    hedge_p = _pad_idx(hedge_idx, PAD_HEDGE)
    zeros1 = jnp.zeros((N_PAD // L, L), f32)

    dvp, dep = _hist_call(node_p, hedge_p, zeros1)
    dvp = dvp.reshape(NW, N_PAD)
    dep = dep.reshape(NW, E_PAD)
    xt, ystack, demat = _tc1_call(feat_p, W_in, b_in.reshape(1, D), W2,
                                  b2.reshape(1, D), dvp, dep)
    pn = _main_call(ystack, demat, node_p, hedge_p)
    return _tc3_call(pn, dvp, xt)
